# baseline (speedup 1.0000x reference)
import functools

import jax
import jax.numpy as jnp
from jax import lax
from jax.experimental import pallas as pl
from jax.experimental.pallas import tpu as pltpu
from jax.experimental.pallas import tpu_sc as plsc

N = 10000
E = 320000
G = 64
H = 128
IN = 16
L = 4
M = 256
NA = 95
EPS = 1e-5

HHALF = H // 2
NTILES = 16
K = 128
WAVE = 4
BODY = 2 * WAVE
CHUNKS = 160
NBODY = CHUNKS // BODY
E_PAD = NTILES * CHUNKS * K
NPAD = 10240
ROWS_PER_TILE = NPAD // NTILES
DEG_PAD = 10112
DEG_ROWS_PER_TILE = DEG_PAD // NTILES
DEG_SPLIT = (CHUNKS + 1) // 2

_MESH = plsc.VectorSubcoreMesh(core_axis_name="c", subcore_axis_name="s")


@functools.partial(
    pl.kernel,
    out_type=jax.ShapeDtypeStruct((2 * DEG_PAD, 16), jnp.float32),
    mesh=_MESH,
    compiler_params=pltpu.CompilerParams(use_tc_tiling_on_sc=False),
    scratch_types=[
        pltpu.VMEM_SHARED((DEG_PAD, 16), jnp.float32),
        pltpu.VMEM((CHUNKS, K), jnp.int32),
        pltpu.VMEM((K, 16), jnp.float32),
        pltpu.SemaphoreType.DMA,
    ],
)
def _sc_deg(dst_hbm, zeros_hbm, ones_hbm, out_hbm, deg_sh, dst_v, ones_v, sem):
    c = lax.axis_index("c")
    s = lax.axis_index("s")
    r0 = s * DEG_ROWS_PER_TILE
    pltpu.sync_copy(zeros_hbm.at[pl.ds(r0, DEG_ROWS_PER_TILE)],
                    deg_sh.at[pl.ds(r0, DEG_ROWS_PER_TILE)])
    pltpu.sync_copy(dst_hbm.at[s], dst_v)
    pltpu.sync_copy(ones_hbm, ones_v)
    plsc.subcore_barrier()
    lo = jnp.where(c == 0, 0, DEG_SPLIT)
    hi = jnp.where(c == 0, DEG_SPLIT, CHUNKS)

    @pl.loop(lo, hi)
    def _chunk(j):
        pltpu.async_copy(ones_v, deg_sh.at[dst_v.at[j]], sem, add=True).wait()

    plsc.subcore_barrier()
    pltpu.sync_copy(deg_sh.at[pl.ds(r0, DEG_ROWS_PER_TILE)],
                    out_hbm.at[pl.ds(c * DEG_PAD + r0, DEG_ROWS_PER_TILE)])


@functools.partial(
    pl.kernel,
    out_type=jax.ShapeDtypeStruct((2 * NPAD, HHALF), jnp.float32),
    mesh=_MESH,
    compiler_params=pltpu.CompilerParams(use_tc_tiling_on_sc=False),
    scratch_types=[
        pltpu.VMEM_SHARED((NPAD, HHALF), jnp.float32),
        pltpu.VMEM((BODY, K), jnp.int32),
        pltpu.VMEM((BODY, K), jnp.int32),
        pltpu.VMEM((WAVE, K, HHALF), jnp.float32),
        [pltpu.SemaphoreType.DMA] * WAVE,
        [pltpu.SemaphoreType.DMA] * WAVE,
    ],
)
def _sc_agg(hw_hbm, src_hbm, dst_hbm, out_hbm,
            acc_sh, src_v, dst_v, rows_v, gsems, ssems):
    c = lax.axis_index("c")
    s = lax.axis_index("s")
    r0 = s * ROWS_PER_TILE
    base = c * NPAD + r0
    pltpu.sync_copy(hw_hbm.at[pl.ds(base, ROWS_PER_TILE)],
                    acc_sh.at[pl.ds(r0, ROWS_PER_TILE)])
    plsc.subcore_barrier()

    def _gather(lj, b):
        return pltpu.async_copy(hw_hbm.at[src_v.at[lj]], rows_v.at[b],
                                gsems[b])

    def _scatter(lj, b):
        return pltpu.async_copy(rows_v.at[b], acc_sh.at[dst_v.at[lj]],
                                ssems[b], add=True)

    @pl.loop(0, NBODY)
    def _body(m):
        pltpu.sync_copy(src_hbm.at[c, s, m], src_v)
        pltpu.sync_copy(dst_hbm.at[s, m], dst_v)
        for b in range(WAVE):
            _gather(b, b)
        for b in range(WAVE):
            pltpu.make_async_copy(hw_hbm.at[src_v.at[b]], rows_v.at[b],
                                  gsems[b]).wait()
            _scatter(b, b)
        for b in range(WAVE):
            pltpu.make_async_copy(rows_v.at[b], acc_sh.at[dst_v.at[b]],
                                  ssems[b]).wait()
            _gather(WAVE + b, b)
        for b in range(WAVE):
            pltpu.make_async_copy(hw_hbm.at[src_v.at[WAVE + b]],
                                  rows_v.at[b], gsems[b]).wait()
            _scatter(WAVE + b, b)
        for b in range(WAVE):
            pltpu.make_async_copy(rows_v.at[b], acc_sh.at[dst_v.at[WAVE + b]],
                                  ssems[b]).wait()

    plsc.subcore_barrier()
    pltpu.sync_copy(acc_sh.at[pl.ds(r0, ROWS_PER_TILE)],
                    out_hbm.at[pl.ds(base, ROWS_PER_TILE)])


def _swish(v):
    return v / (1.0 + jnp.exp(-v))


def _tc_embed_body(x_ref, zemb_ref, ew_ref, eb_ref, deg_ref, w0_ref,
                   h_ref, dinv_ref, hw_ref):
    x = x_ref[...]
    z = x[:, 0:1].astype(jnp.int32)
    onehot = (z == lax.broadcasted_iota(jnp.int32, (N, NA), 1)).astype(jnp.float32)
    h = jnp.dot(onehot, zemb_ref[...], preferred_element_type=jnp.float32)
    h = h + jnp.dot(x[:, 1:], ew_ref[...].T, preferred_element_type=jnp.float32)
    h = h + eb_ref[...]
    deg = deg_ref[0:N, 0:1] + deg_ref[DEG_PAD:DEG_PAD + N, 0:1] + 1.0
    dinv = lax.rsqrt(deg)
    h_ref[...] = h
    dinv_ref[...] = dinv
    hw = jnp.dot(h, w0_ref[...].T, preferred_element_type=jnp.float32) * dinv
    hw_ref[0, :N] = hw[:, :HHALF]
    hw_ref[1, :N] = hw[:, HHALF:]


def _layer_math(acc_ref, dinv_ref, hres_ref, batch_ref, cb_ref, gnw_ref,
                gnb_ref, gnms_ref, w1_ref, b1_ref, w2_ref, b2_ref):
    dinv = dinv_ref[...]
    hcat = jnp.concatenate([acc_ref[0, :N], acc_ref[1, :N]], axis=1)
    h1 = dinv * hcat + cb_ref[...]
    seg = (batch_ref[...].astype(jnp.int32)
           == lax.broadcasted_iota(jnp.int32, (G, N), 0))
    seg = seg.astype(jnp.float32)
    cnt = jnp.maximum(jnp.sum(seg, axis=1, keepdims=True), 1.0)
    mean = jnp.dot(seg, h1, preferred_element_type=jnp.float32) / cnt
    mean_n = lax.dot_general(seg, mean, (((0,), (0,)), ((), ())),
                             preferred_element_type=jnp.float32)
    out_c = h1 - mean_n * gnms_ref[...]
    var = jnp.dot(seg, out_c * out_c, preferred_element_type=jnp.float32) / cnt
    var_n = lax.dot_general(seg, var, (((0,), (0,)), ((), ())),
                            preferred_element_type=jnp.float32)
    h2 = gnw_ref[...] * out_c / jnp.sqrt(var_n + EPS) + gnb_ref[...]
    h3 = _swish(h2)
    m = _swish(jnp.dot(h3, w1_ref[...].T, preferred_element_type=jnp.float32)
               + b1_ref[...])
    m = _swish(jnp.dot(m, w2_ref[...].T, preferred_element_type=jnp.float32)
               + b2_ref[...])
    return m + hres_ref[...]


def _tc_layer_body(acc_ref, dinv_ref, hres_ref, batch_ref, cb_ref, gnw_ref,
                   gnb_ref, gnms_ref, w1_ref, b1_ref, w2_ref, b2_ref, wnext_ref,
                   h_ref, hw_ref):
    h_out = _layer_math(acc_ref, dinv_ref, hres_ref, batch_ref, cb_ref, gnw_ref,
                        gnb_ref, gnms_ref, w1_ref, b1_ref, w2_ref, b2_ref)
    h_ref[...] = h_out
    hw = jnp.dot(h_out, wnext_ref[...].T,
                 preferred_element_type=jnp.float32) * dinv_ref[...]
    hw_ref[0, :N] = hw[:, :HHALF]
    hw_ref[1, :N] = hw[:, HHALF:]


def _tc_final_body(acc_ref, dinv_ref, hres_ref, batch_ref, cb_ref, gnw_ref,
                   gnb_ref, gnms_ref, w1_ref, b1_ref, w2_ref, b2_ref, h_ref):
    h_ref[...] = _layer_math(acc_ref, dinv_ref, hres_ref, batch_ref, cb_ref,
                             gnw_ref, gnb_ref, gnms_ref, w1_ref, b1_ref,
                             w2_ref, b2_ref)


_f32 = jnp.float32

_tc_embed = pl.pallas_call(
    _tc_embed_body,
    out_shape=[
        jax.ShapeDtypeStruct((N, H), _f32),
        jax.ShapeDtypeStruct((N, 1), _f32),
        jax.ShapeDtypeStruct((2, NPAD, HHALF), _f32),
    ],
)

_tc_layer = pl.pallas_call(
    _tc_layer_body,
    out_shape=[
        jax.ShapeDtypeStruct((N, H), _f32),
        jax.ShapeDtypeStruct((2, NPAD, HHALF), _f32),
    ],
)

_tc_final = pl.pallas_call(
    _tc_final_body,
    out_shape=jax.ShapeDtypeStruct((N, H), _f32),
)


def kernel(x, edge_index, batch, z_embed, extra_W, extra_b, conv_W, conv_b,
           gn_w, gn_b, gn_ms, mlp_W1, mlp_b1, mlp_W2, mlp_b2):
    src = edge_index[0]
    dst = edge_index[1]
    pad = E_PAD - E
    src_t = jnp.concatenate([src, jnp.zeros((pad,), jnp.int32)]).reshape(
        NTILES, NBODY, BODY, K)
    src_2 = jnp.stack([src_t, src_t + NPAD])
    dst_t = jnp.concatenate([dst, jnp.full((pad,), N, jnp.int32)]).reshape(
        NTILES, NBODY, BODY, K)

    zeros_deg = jnp.zeros((DEG_PAD, 16), _f32)
    ones_k = jnp.ones((K, 16), _f32)
    deg16 = _sc_deg(dst_t.reshape(NTILES, CHUNKS, K), zeros_deg, ones_k)

    batch_row = batch.astype(_f32).reshape(1, N)
    h, dinv, hw = _tc_embed(x, z_embed, extra_W, extra_b.reshape(1, H),
                            deg16, conv_W[0])

    for i in range(L):
        acc = _sc_agg(hw.reshape(2 * NPAD, HHALF), src_2, dst_t)
        args = (acc.reshape(2, NPAD, HHALF), dinv, h, batch_row,
                conv_b[i].reshape(1, H), gn_w[i].reshape(1, H),
                gn_b[i].reshape(1, H), gn_ms[i].reshape(1, H),
                mlp_W1[i], mlp_b1[i].reshape(1, M),
                mlp_W2[i], mlp_b2[i].reshape(1, H))
        if i + 1 < L:
            h, hw = _tc_layer(*args, conv_W[i + 1])
        else:
            h = _tc_final(*args)
    return h


# speedup vs baseline: 1.2700x; 1.2700x over previous
import functools

import jax
import jax.numpy as jnp
from jax import lax
from jax.experimental import pallas as pl
from jax.experimental.pallas import tpu as pltpu
from jax.experimental.pallas import tpu_sc as plsc

N = 10000
E = 320000
G = 64
H = 128
IN = 16
L = 4
M = 256
NA = 95
EPS = 1e-5

HHALF = H // 2
NTILES = 16
K = 128
WAVE = 4
BODY = 2 * WAVE
CHUNKS = 160
NBODY = CHUNKS // BODY
E_PAD = NTILES * CHUNKS * K
NPAD = 10240
ROWS_PER_TILE = NPAD // NTILES
DEG_PAD = 10112
DEG_ROWS_PER_TILE = DEG_PAD // NTILES
DEG_SPLIT = (CHUNKS + 1) // 2

_MESH = plsc.VectorSubcoreMesh(core_axis_name="c", subcore_axis_name="s")


@functools.partial(
    pl.kernel,
    out_type=jax.ShapeDtypeStruct((2 * DEG_PAD, 16), jnp.float32),
    mesh=_MESH,
    compiler_params=pltpu.CompilerParams(use_tc_tiling_on_sc=False),
    scratch_types=[
        pltpu.VMEM_SHARED((DEG_PAD, 16), jnp.float32),
        pltpu.VMEM((CHUNKS, K), jnp.int32),
        pltpu.VMEM((K, 16), jnp.float32),
        pltpu.SemaphoreType.DMA,
    ],
)
def _sc_deg(dst_hbm, zeros_hbm, ones_hbm, out_hbm, deg_sh, dst_v, ones_v, sem):
    c = lax.axis_index("c")
    s = lax.axis_index("s")
    r0 = s * DEG_ROWS_PER_TILE
    pltpu.sync_copy(zeros_hbm.at[pl.ds(r0, DEG_ROWS_PER_TILE)],
                    deg_sh.at[pl.ds(r0, DEG_ROWS_PER_TILE)])
    pltpu.sync_copy(dst_hbm.at[s], dst_v)
    pltpu.sync_copy(ones_hbm, ones_v)
    plsc.subcore_barrier()
    lo = jnp.where(c == 0, 0, DEG_SPLIT)
    hi = jnp.where(c == 0, DEG_SPLIT, CHUNKS)

    @pl.loop(lo, hi)
    def _chunk(j):
        pltpu.async_copy(ones_v, deg_sh.at[dst_v.at[j]], sem, add=True).wait()

    plsc.subcore_barrier()
    pltpu.sync_copy(deg_sh.at[pl.ds(r0, DEG_ROWS_PER_TILE)],
                    out_hbm.at[pl.ds(c * DEG_PAD + r0, DEG_ROWS_PER_TILE)])


@functools.partial(
    pl.kernel,
    out_type=jax.ShapeDtypeStruct((2 * NPAD, HHALF), jnp.float32),
    mesh=_MESH,
    compiler_params=pltpu.CompilerParams(use_tc_tiling_on_sc=False),
    scratch_types=[
        pltpu.VMEM_SHARED((NPAD, HHALF), jnp.float32),
        pltpu.VMEM_SHARED((NPAD, HHALF), jnp.float32),
        pltpu.VMEM((BODY, K), jnp.int32),
        pltpu.VMEM((BODY, K), jnp.int32),
        pltpu.VMEM((BODY, K), jnp.int32),
        pltpu.VMEM((WAVE, K, HHALF), jnp.float32),
        [pltpu.SemaphoreType.DMA] * WAVE,
        [pltpu.SemaphoreType.DMA] * WAVE,
    ],
)
def _sc_agg(hw_hbm, src_hbm, src2_hbm, dst_hbm, out_hbm,
            table_sh, acc_sh, src_v, src2_v, dst_v, rows_v, gsems, ssems):
    c = lax.axis_index("c")
    s = lax.axis_index("s")
    r0 = s * ROWS_PER_TILE
    base = c * NPAD + r0
    pltpu.sync_copy(hw_hbm.at[pl.ds(base, ROWS_PER_TILE)],
                    table_sh.at[pl.ds(r0, ROWS_PER_TILE)])
    pltpu.sync_copy(hw_hbm.at[pl.ds(base, ROWS_PER_TILE)],
                    acc_sh.at[pl.ds(r0, ROWS_PER_TILE)])
    plsc.subcore_barrier()

    def _gather(lj, b):
        if b % 2 == 0:
            return pltpu.async_copy(table_sh.at[src_v.at[lj]], rows_v.at[b],
                                    gsems[b])
        return pltpu.async_copy(hw_hbm.at[src2_v.at[lj]], rows_v.at[b],
                                gsems[b])

    def _wait_gather(lj, b):
        if b % 2 == 0:
            pltpu.make_async_copy(table_sh.at[src_v.at[lj]], rows_v.at[b],
                                  gsems[b]).wait()
        else:
            pltpu.make_async_copy(hw_hbm.at[src2_v.at[lj]], rows_v.at[b],
                                  gsems[b]).wait()

    def _scatter(lj, b):
        return pltpu.async_copy(rows_v.at[b], acc_sh.at[dst_v.at[lj]],
                                ssems[b], add=True)

    @pl.loop(0, NBODY)
    def _body(m):
        pltpu.sync_copy(src_hbm.at[s, m], src_v)
        pltpu.sync_copy(src2_hbm.at[c, s, m], src2_v)
        pltpu.sync_copy(dst_hbm.at[s, m], dst_v)
        for b in range(WAVE):
            _gather(b, b)
        for b in range(WAVE):
            _wait_gather(b, b)
            _scatter(b, b)
        for b in range(WAVE):
            pltpu.make_async_copy(rows_v.at[b], acc_sh.at[dst_v.at[b]],
                                  ssems[b]).wait()
            _gather(WAVE + b, b)
        for b in range(WAVE):
            _wait_gather(WAVE + b, b)
            _scatter(WAVE + b, b)
        for b in range(WAVE):
            pltpu.make_async_copy(rows_v.at[b], acc_sh.at[dst_v.at[WAVE + b]],
                                  ssems[b]).wait()

    plsc.subcore_barrier()
    pltpu.sync_copy(acc_sh.at[pl.ds(r0, ROWS_PER_TILE)],
                    out_hbm.at[pl.ds(base, ROWS_PER_TILE)])


def _swish(v):
    return v / (1.0 + jnp.exp(-v))


def _tc_embed_body(x_ref, zemb_ref, ew_ref, eb_ref, deg_ref, w0_ref,
                   h_ref, dinv_ref, hw_ref):
    x = x_ref[...]
    z = x[:, 0:1].astype(jnp.int32)
    onehot = (z == lax.broadcasted_iota(jnp.int32, (N, NA), 1)).astype(jnp.float32)
    h = jnp.dot(onehot, zemb_ref[...], preferred_element_type=jnp.float32)
    h = h + jnp.dot(x[:, 1:], ew_ref[...].T, preferred_element_type=jnp.float32)
    h = h + eb_ref[...]
    deg = deg_ref[0:N, 0:1] + deg_ref[DEG_PAD:DEG_PAD + N, 0:1] + 1.0
    dinv = lax.rsqrt(deg)
    h_ref[...] = h
    dinv_ref[...] = dinv
    hw = jnp.dot(h, w0_ref[...].T, preferred_element_type=jnp.float32) * dinv
    hw_ref[0, :N] = hw[:, :HHALF]
    hw_ref[1, :N] = hw[:, HHALF:]


def _layer_math(acc_ref, dinv_ref, hres_ref, batch_ref, cb_ref, gnw_ref,
                gnb_ref, gnms_ref, w1_ref, b1_ref, w2_ref, b2_ref):
    dinv = dinv_ref[...]
    hcat = jnp.concatenate([acc_ref[0, :N], acc_ref[1, :N]], axis=1)
    h1 = dinv * hcat + cb_ref[...]
    seg = (batch_ref[...].astype(jnp.int32)
           == lax.broadcasted_iota(jnp.int32, (G, N), 0))
    seg = seg.astype(jnp.float32)
    cnt = jnp.maximum(jnp.sum(seg, axis=1, keepdims=True), 1.0)
    mean = jnp.dot(seg, h1, preferred_element_type=jnp.float32) / cnt
    mean_n = lax.dot_general(seg, mean, (((0,), (0,)), ((), ())),
                             preferred_element_type=jnp.float32)
    out_c = h1 - mean_n * gnms_ref[...]
    var = jnp.dot(seg, out_c * out_c, preferred_element_type=jnp.float32) / cnt
    var_n = lax.dot_general(seg, var, (((0,), (0,)), ((), ())),
                            preferred_element_type=jnp.float32)
    h2 = gnw_ref[...] * out_c / jnp.sqrt(var_n + EPS) + gnb_ref[...]
    h3 = _swish(h2)
    m = _swish(jnp.dot(h3, w1_ref[...].T, preferred_element_type=jnp.float32)
               + b1_ref[...])
    m = _swish(jnp.dot(m, w2_ref[...].T, preferred_element_type=jnp.float32)
               + b2_ref[...])
    return m + hres_ref[...]


def _tc_layer_body(acc_ref, dinv_ref, hres_ref, batch_ref, cb_ref, gnw_ref,
                   gnb_ref, gnms_ref, w1_ref, b1_ref, w2_ref, b2_ref, wnext_ref,
                   h_ref, hw_ref):
    h_out = _layer_math(acc_ref, dinv_ref, hres_ref, batch_ref, cb_ref, gnw_ref,
                        gnb_ref, gnms_ref, w1_ref, b1_ref, w2_ref, b2_ref)
    h_ref[...] = h_out
    hw = jnp.dot(h_out, wnext_ref[...].T,
                 preferred_element_type=jnp.float32) * dinv_ref[...]
    hw_ref[0, :N] = hw[:, :HHALF]
    hw_ref[1, :N] = hw[:, HHALF:]


def _tc_final_body(acc_ref, dinv_ref, hres_ref, batch_ref, cb_ref, gnw_ref,
                   gnb_ref, gnms_ref, w1_ref, b1_ref, w2_ref, b2_ref, h_ref):
    h_ref[...] = _layer_math(acc_ref, dinv_ref, hres_ref, batch_ref, cb_ref,
                             gnw_ref, gnb_ref, gnms_ref, w1_ref, b1_ref,
                             w2_ref, b2_ref)


_f32 = jnp.float32

_tc_embed = pl.pallas_call(
    _tc_embed_body,
    out_shape=[
        jax.ShapeDtypeStruct((N, H), _f32),
        jax.ShapeDtypeStruct((N, 1), _f32),
        jax.ShapeDtypeStruct((2, NPAD, HHALF), _f32),
    ],
)

_tc_layer = pl.pallas_call(
    _tc_layer_body,
    out_shape=[
        jax.ShapeDtypeStruct((N, H), _f32),
        jax.ShapeDtypeStruct((2, NPAD, HHALF), _f32),
    ],
)

_tc_final = pl.pallas_call(
    _tc_final_body,
    out_shape=jax.ShapeDtypeStruct((N, H), _f32),
)


def kernel(x, edge_index, batch, z_embed, extra_W, extra_b, conv_W, conv_b,
           gn_w, gn_b, gn_ms, mlp_W1, mlp_b1, mlp_W2, mlp_b2):
    src = edge_index[0]
    dst = edge_index[1]
    pad = E_PAD - E
    src_t = jnp.concatenate([src, jnp.zeros((pad,), jnp.int32)]).reshape(
        NTILES, NBODY, BODY, K)
    src_2 = jnp.stack([src_t, src_t + NPAD])
    dst_t = jnp.concatenate([dst, jnp.full((pad,), N, jnp.int32)]).reshape(
        NTILES, NBODY, BODY, K)

    zeros_deg = jnp.zeros((DEG_PAD, 16), _f32)
    ones_k = jnp.ones((K, 16), _f32)
    deg16 = _sc_deg(dst_t.reshape(NTILES, CHUNKS, K), zeros_deg, ones_k)

    batch_row = batch.astype(_f32).reshape(1, N)
    h, dinv, hw = _tc_embed(x, z_embed, extra_W, extra_b.reshape(1, H),
                            deg16, conv_W[0])

    for i in range(L):
        acc = _sc_agg(hw.reshape(2 * NPAD, HHALF), src_t, src_2, dst_t)
        args = (acc.reshape(2, NPAD, HHALF), dinv, h, batch_row,
                conv_b[i].reshape(1, H), gn_w[i].reshape(1, H),
                gn_b[i].reshape(1, H), gn_ms[i].reshape(1, H),
                mlp_W1[i], mlp_b1[i].reshape(1, M),
                mlp_W2[i], mlp_b2[i].reshape(1, H))
        if i + 1 < L:
            h, hw = _tc_layer(*args, conv_W[i + 1])
        else:
            h = _tc_final(*args)
    return h


# speedup vs baseline: 1.6931x; 1.3332x over previous
import functools

import jax
import jax.numpy as jnp
from jax import lax
from jax.experimental import pallas as pl
from jax.experimental.pallas import tpu as pltpu
from jax.experimental.pallas import tpu_sc as plsc

N = 10000
E = 320000
G = 64
H = 128
IN = 16
L = 4
M = 256
NA = 95
EPS = 1e-5

HHALF = H // 2
NTILES = 16
K = 128
WAVE = 4
BODY = 2 * WAVE
CHUNKS = 160
NBODY = CHUNKS // BODY
E_PAD = NTILES * CHUNKS * K
NPAD = 10240
ROWS_PER_TILE = NPAD // NTILES
DEG_PAD = 10112
DEG_ROWS_PER_TILE = DEG_PAD // NTILES
DEG_SPLIT = (CHUNKS + 1) // 2

_MESH = plsc.VectorSubcoreMesh(core_axis_name="c", subcore_axis_name="s")


@functools.partial(
    pl.kernel,
    out_type=jax.ShapeDtypeStruct((2 * DEG_PAD, 16), jnp.float32),
    mesh=_MESH,
    compiler_params=pltpu.CompilerParams(use_tc_tiling_on_sc=False),
    scratch_types=[
        pltpu.VMEM_SHARED((DEG_PAD, 16), jnp.float32),
        pltpu.VMEM((CHUNKS, K), jnp.int32),
        pltpu.VMEM((K, 16), jnp.float32),
        pltpu.SemaphoreType.DMA,
    ],
)
def _sc_deg(dst_hbm, zeros_hbm, ones_hbm, out_hbm, deg_sh, dst_v, ones_v, sem):
    c = lax.axis_index("c")
    s = lax.axis_index("s")
    r0 = s * DEG_ROWS_PER_TILE
    pltpu.sync_copy(zeros_hbm.at[pl.ds(r0, DEG_ROWS_PER_TILE)],
                    deg_sh.at[pl.ds(r0, DEG_ROWS_PER_TILE)])
    pltpu.sync_copy(dst_hbm.at[s], dst_v)
    pltpu.sync_copy(ones_hbm, ones_v)
    plsc.subcore_barrier()
    lo = jnp.where(c == 0, 0, DEG_SPLIT)
    hi = jnp.where(c == 0, DEG_SPLIT, CHUNKS)

    @pl.loop(lo, hi, step=16)
    def _chunk(j0):
        for t in range(16):
            pltpu.async_copy(ones_v, deg_sh.at[dst_v.at[j0 + t]], sem,
                             add=True)
        for t in range(16):
            pltpu.make_async_copy(ones_v, deg_sh.at[dst_v.at[j0 + t]],
                                  sem).wait()

    plsc.subcore_barrier()
    pltpu.sync_copy(deg_sh.at[pl.ds(r0, DEG_ROWS_PER_TILE)],
                    out_hbm.at[pl.ds(c * DEG_PAD + r0, DEG_ROWS_PER_TILE)])


@functools.partial(
    pl.kernel,
    out_type=jax.ShapeDtypeStruct((2 * NPAD, HHALF), jnp.float32),
    mesh=_MESH,
    compiler_params=pltpu.CompilerParams(use_tc_tiling_on_sc=False),
    scratch_types=[
        pltpu.VMEM_SHARED((NPAD, HHALF), jnp.float32),
        pltpu.VMEM_SHARED((NPAD, HHALF), jnp.float32),
        pltpu.VMEM((2, BODY, K), jnp.int32),
        pltpu.VMEM((2, BODY, K), jnp.int32),
        pltpu.VMEM((WAVE, K, HHALF), jnp.float32),
        [pltpu.SemaphoreType.DMA] * WAVE,
        [pltpu.SemaphoreType.DMA] * WAVE,
        pltpu.SemaphoreType.DMA,
        pltpu.SemaphoreType.DMA,
    ],
)
def _sc_agg(hw_hbm, idx_hbm, out_hbm,
            table_sh, acc_sh, idx_a, idx_b, rows_v, gsems, ssems,
            isem_a, isem_b):
    c = lax.axis_index("c")
    s = lax.axis_index("s")
    r0 = s * ROWS_PER_TILE
    base = c * NPAD + r0
    cp_t = pltpu.async_copy(hw_hbm.at[pl.ds(base, ROWS_PER_TILE)],
                            table_sh.at[pl.ds(r0, ROWS_PER_TILE)], gsems[0])
    cp_a = pltpu.async_copy(hw_hbm.at[pl.ds(base, ROWS_PER_TILE)],
                            acc_sh.at[pl.ds(r0, ROWS_PER_TILE)], gsems[1])
    pltpu.sync_copy(idx_hbm.at[s, 0], idx_a)
    pltpu.async_copy(idx_hbm.at[s, 1], idx_b, isem_b)
    cp_t.wait()
    cp_a.wait()
    plsc.subcore_barrier()

    def _run_body(idx_v):
        def _gather(lj, b):
            return pltpu.async_copy(table_sh.at[idx_v.at[0, lj]],
                                    rows_v.at[b], gsems[b])

        def _wait_gather(lj, b):
            pltpu.make_async_copy(table_sh.at[idx_v.at[0, lj]], rows_v.at[b],
                                  gsems[b]).wait()

        def _scatter(lj, b):
            return pltpu.async_copy(rows_v.at[b], acc_sh.at[idx_v.at[1, lj]],
                                    ssems[b], add=True)

        def _wait_scatter(lj, b):
            pltpu.make_async_copy(rows_v.at[b], acc_sh.at[idx_v.at[1, lj]],
                                  ssems[b]).wait()

        for b in range(WAVE):
            _gather(b, b)
        for b in range(WAVE):
            _wait_gather(b, b)
            _scatter(b, b)
        for b in range(WAVE):
            _wait_scatter(b, b)
            _gather(WAVE + b, b)
        for b in range(WAVE):
            _wait_gather(WAVE + b, b)
            _scatter(WAVE + b, b)
        for b in range(WAVE):
            _wait_scatter(WAVE + b, b)

    @pl.loop(0, NBODY, step=2)
    def _body(m):
        @pl.when(m > 0)
        def _wait_idx_a():
            pltpu.make_async_copy(idx_hbm.at[s, m], idx_a, isem_a).wait()

        _run_body(idx_a)

        @pl.when(m + 2 < NBODY)
        def _prefetch_a():
            pltpu.async_copy(idx_hbm.at[s, m + 2], idx_a, isem_a)

        pltpu.make_async_copy(idx_hbm.at[s, m + 1], idx_b, isem_b).wait()
        _run_body(idx_b)

        @pl.when(m + 3 < NBODY)
        def _prefetch_b():
            pltpu.async_copy(idx_hbm.at[s, m + 3], idx_b, isem_b)

    plsc.subcore_barrier()
    pltpu.sync_copy(acc_sh.at[pl.ds(r0, ROWS_PER_TILE)],
                    out_hbm.at[pl.ds(base, ROWS_PER_TILE)])


def _swish(v):
    return v / (1.0 + jnp.exp(-v))


def _tc_embed_body(x_ref, zemb_ref, ew_ref, eb_ref, deg_ref, w0_ref,
                   h_ref, dinv_ref, hw_ref):
    x = x_ref[...]
    z = x[:, 0:1].astype(jnp.int32)
    onehot = (z == lax.broadcasted_iota(jnp.int32, (N, NA), 1)).astype(jnp.float32)
    h = jnp.dot(onehot, zemb_ref[...], preferred_element_type=jnp.float32)
    h = h + jnp.dot(x[:, 1:], ew_ref[...].T, preferred_element_type=jnp.float32)
    h = h + eb_ref[...]
    deg = deg_ref[0:N, 0:1] + deg_ref[DEG_PAD:DEG_PAD + N, 0:1] + 1.0
    dinv = lax.rsqrt(deg)
    h_ref[...] = h
    dinv_ref[...] = dinv
    hw = jnp.dot(h, w0_ref[...].T, preferred_element_type=jnp.float32) * dinv
    hw_ref[0, :N] = hw[:, :HHALF]
    hw_ref[1, :N] = hw[:, HHALF:]


def _layer_math(acc_ref, dinv_ref, hres_ref, batch_ref, cb_ref, gnw_ref,
                gnb_ref, gnms_ref, w1_ref, b1_ref, w2_ref, b2_ref):
    dinv = dinv_ref[...]
    hcat = jnp.concatenate([acc_ref[0, :N], acc_ref[1, :N]], axis=1)
    h1 = dinv * hcat + cb_ref[...]
    seg = (batch_ref[...].astype(jnp.int32)
           == lax.broadcasted_iota(jnp.int32, (G, N), 0))
    seg = seg.astype(jnp.float32)
    cnt = jnp.maximum(jnp.sum(seg, axis=1, keepdims=True), 1.0)
    mean = jnp.dot(seg, h1, preferred_element_type=jnp.float32) / cnt
    mean_n = lax.dot_general(seg, mean, (((0,), (0,)), ((), ())),
                             preferred_element_type=jnp.float32)
    out_c = h1 - mean_n * gnms_ref[...]
    var = jnp.dot(seg, out_c * out_c, preferred_element_type=jnp.float32) / cnt
    var_n = lax.dot_general(seg, var, (((0,), (0,)), ((), ())),
                            preferred_element_type=jnp.float32)
    h2 = gnw_ref[...] * out_c / jnp.sqrt(var_n + EPS) + gnb_ref[...]
    h3 = _swish(h2)
    m = _swish(jnp.dot(h3, w1_ref[...].T, preferred_element_type=jnp.float32)
               + b1_ref[...])
    m = _swish(jnp.dot(m, w2_ref[...].T, preferred_element_type=jnp.float32)
               + b2_ref[...])
    return m + hres_ref[...]


def _tc_layer_body(acc_ref, dinv_ref, hres_ref, batch_ref, cb_ref, gnw_ref,
                   gnb_ref, gnms_ref, w1_ref, b1_ref, w2_ref, b2_ref, wnext_ref,
                   h_ref, hw_ref):
    h_out = _layer_math(acc_ref, dinv_ref, hres_ref, batch_ref, cb_ref, gnw_ref,
                        gnb_ref, gnms_ref, w1_ref, b1_ref, w2_ref, b2_ref)
    h_ref[...] = h_out
    hw = jnp.dot(h_out, wnext_ref[...].T,
                 preferred_element_type=jnp.float32) * dinv_ref[...]
    hw_ref[0, :N] = hw[:, :HHALF]
    hw_ref[1, :N] = hw[:, HHALF:]


def _tc_final_body(acc_ref, dinv_ref, hres_ref, batch_ref, cb_ref, gnw_ref,
                   gnb_ref, gnms_ref, w1_ref, b1_ref, w2_ref, b2_ref, h_ref):
    h_ref[...] = _layer_math(acc_ref, dinv_ref, hres_ref, batch_ref, cb_ref,
                             gnw_ref, gnb_ref, gnms_ref, w1_ref, b1_ref,
                             w2_ref, b2_ref)


_f32 = jnp.float32

_tc_embed = pl.pallas_call(
    _tc_embed_body,
    out_shape=[
        jax.ShapeDtypeStruct((N, H), _f32),
        jax.ShapeDtypeStruct((N, 1), _f32),
        jax.ShapeDtypeStruct((2, NPAD, HHALF), _f32),
    ],
)

_tc_layer = pl.pallas_call(
    _tc_layer_body,
    out_shape=[
        jax.ShapeDtypeStruct((N, H), _f32),
        jax.ShapeDtypeStruct((2, NPAD, HHALF), _f32),
    ],
)

_tc_final = pl.pallas_call(
    _tc_final_body,
    out_shape=jax.ShapeDtypeStruct((N, H), _f32),
)


def kernel(x, edge_index, batch, z_embed, extra_W, extra_b, conv_W, conv_b,
           gn_w, gn_b, gn_ms, mlp_W1, mlp_b1, mlp_W2, mlp_b2):
    src = edge_index[0]
    dst = edge_index[1]
    pad = E_PAD - E
    src_t = jnp.concatenate([src, jnp.zeros((pad,), jnp.int32)]).reshape(
        NTILES, NBODY, BODY, K)
    dst_t = jnp.concatenate([dst, jnp.full((pad,), N, jnp.int32)]).reshape(
        NTILES, NBODY, BODY, K)
    idx_t = jnp.stack([src_t, dst_t], axis=2)

    zeros_deg = jnp.zeros((DEG_PAD, 16), _f32)
    ones_k = jnp.ones((K, 16), _f32)
    deg16 = _sc_deg(dst_t.reshape(NTILES, CHUNKS, K), zeros_deg, ones_k)

    batch_row = batch.astype(_f32).reshape(1, N)
    h, dinv, hw = _tc_embed(x, z_embed, extra_W, extra_b.reshape(1, H),
                            deg16, conv_W[0])

    for i in range(L):
        acc = _sc_agg(hw.reshape(2 * NPAD, HHALF), idx_t)
        args = (acc.reshape(2, NPAD, HHALF), dinv, h, batch_row,
                conv_b[i].reshape(1, H), gn_w[i].reshape(1, H),
                gn_b[i].reshape(1, H), gn_ms[i].reshape(1, H),
                mlp_W1[i], mlp_b1[i].reshape(1, M),
                mlp_W2[i], mlp_b2[i].reshape(1, H))
        if i + 1 < L:
            h, hw = _tc_layer(*args, conv_W[i + 1])
        else:
            h = _tc_final(*args)
    return h


# speedup vs baseline: 1.7880x; 1.0560x over previous
import functools

import jax
import jax.numpy as jnp
from jax import lax
from jax.experimental import pallas as pl
from jax.experimental.pallas import tpu as pltpu
from jax.experimental.pallas import tpu_sc as plsc

N = 10000
E = 320000
G = 64
H = 128
IN = 16
L = 4
M = 256
NA = 95
EPS = 1e-5

HHALF = H // 2
NTILES = 16
K = 128
WAVE = 4
BODY = 2 * WAVE
CHUNKS = 160
NBODY = CHUNKS // BODY
E_PAD = NTILES * CHUNKS * K
NPAD = 10240
ROWS_PER_TILE = NPAD // NTILES
DEG_PAD = 10112
DEG_ROWS_PER_TILE = DEG_PAD // NTILES
DEG_SPLIT = (CHUNKS + 1) // 2

_MESH = plsc.VectorSubcoreMesh(core_axis_name="c", subcore_axis_name="s")


@functools.partial(
    pl.kernel,
    out_type=jax.ShapeDtypeStruct((2 * DEG_PAD, 16), jnp.float32),
    mesh=_MESH,
    compiler_params=pltpu.CompilerParams(use_tc_tiling_on_sc=False),
    scratch_types=[
        pltpu.VMEM_SHARED((DEG_PAD, 16), jnp.float32),
        pltpu.VMEM((CHUNKS, K), jnp.int32),
        pltpu.VMEM((K, 16), jnp.float32),
        pltpu.SemaphoreType.DMA,
    ],
)
def _sc_deg(dst_hbm, zeros_hbm, ones_hbm, out_hbm, deg_sh, dst_v, ones_v, sem):
    c = lax.axis_index("c")
    s = lax.axis_index("s")
    r0 = s * DEG_ROWS_PER_TILE
    pltpu.sync_copy(zeros_hbm.at[pl.ds(r0, DEG_ROWS_PER_TILE)],
                    deg_sh.at[pl.ds(r0, DEG_ROWS_PER_TILE)])
    pltpu.sync_copy(dst_hbm.at[s], dst_v)
    pltpu.sync_copy(ones_hbm, ones_v)
    plsc.subcore_barrier()
    lo = jnp.where(c == 0, 0, DEG_SPLIT)
    hi = jnp.where(c == 0, DEG_SPLIT, CHUNKS)

    @pl.loop(lo, hi, step=16)
    def _chunk(j0):
        for t in range(16):
            pltpu.async_copy(ones_v, deg_sh.at[dst_v.at[j0 + t]], sem,
                             add=True)
        for t in range(16):
            pltpu.make_async_copy(ones_v, deg_sh.at[dst_v.at[j0 + t]],
                                  sem).wait()

    plsc.subcore_barrier()
    pltpu.sync_copy(deg_sh.at[pl.ds(r0, DEG_ROWS_PER_TILE)],
                    out_hbm.at[pl.ds(c * DEG_PAD + r0, DEG_ROWS_PER_TILE)])


@functools.partial(
    pl.kernel,
    out_type=jax.ShapeDtypeStruct((2 * NPAD, HHALF), jnp.float32),
    mesh=_MESH,
    compiler_params=pltpu.CompilerParams(use_tc_tiling_on_sc=False),
    scratch_types=[
        pltpu.VMEM_SHARED((NPAD, HHALF), jnp.float32),
        pltpu.VMEM_SHARED((NPAD, HHALF), jnp.float32),
        pltpu.VMEM((2, BODY, K), jnp.int32),
        pltpu.VMEM((2, BODY, K), jnp.int32),
        pltpu.VMEM((WAVE, K, HHALF), jnp.float32),
        [pltpu.SemaphoreType.DMA] * WAVE,
        [pltpu.SemaphoreType.DMA] * WAVE,
        pltpu.SemaphoreType.DMA,
        pltpu.SemaphoreType.DMA,
    ],
)
def _sc_agg(hw_hbm, idx_hbm, out_hbm,
            table_sh, acc_sh, idx_a, idx_b, rows_v, gsems, ssems,
            isem_a, isem_b):
    c = lax.axis_index("c")
    s = lax.axis_index("s")
    r0 = s * ROWS_PER_TILE
    base = c * NPAD + r0
    cp_t = pltpu.async_copy(hw_hbm.at[pl.ds(base, ROWS_PER_TILE)],
                            table_sh.at[pl.ds(r0, ROWS_PER_TILE)], gsems[0])
    cp_a = pltpu.async_copy(hw_hbm.at[pl.ds(base, ROWS_PER_TILE)],
                            acc_sh.at[pl.ds(r0, ROWS_PER_TILE)], gsems[1])
    cp_t.wait()
    cp_a.wait()
    plsc.subcore_barrier()

    def _run_body(idx_v):
        def _gather(lj, b):
            return pltpu.async_copy(table_sh.at[idx_v.at[0, lj]],
                                    rows_v.at[b], gsems[b])

        def _wait_gather(lj, b):
            pltpu.make_async_copy(table_sh.at[idx_v.at[0, lj]], rows_v.at[b],
                                  gsems[b]).wait()

        def _scatter(lj, b):
            return pltpu.async_copy(rows_v.at[b], acc_sh.at[idx_v.at[1, lj]],
                                    ssems[b], add=True)

        def _wait_scatter(lj, b):
            pltpu.make_async_copy(rows_v.at[b], acc_sh.at[idx_v.at[1, lj]],
                                  ssems[b]).wait()

        for b in range(WAVE):
            _gather(b, b)
        for b in range(WAVE):
            _wait_gather(b, b)
            _scatter(b, b)
        for b in range(WAVE):
            _wait_scatter(b, b)
            _gather(WAVE + b, b)
        for b in range(WAVE):
            _wait_gather(WAVE + b, b)
            _scatter(WAVE + b, b)
        for b in range(WAVE):
            _wait_scatter(WAVE + b, b)

    @pl.loop(0, NBODY)
    def _body(m):
        pltpu.sync_copy(idx_hbm.at[s, m], idx_a)
        _run_body(idx_a)

    plsc.subcore_barrier()
    pltpu.sync_copy(acc_sh.at[pl.ds(r0, ROWS_PER_TILE)],
                    out_hbm.at[pl.ds(base, ROWS_PER_TILE)])


def _swish(v):
    return v / (1.0 + jnp.exp(-v))


def _tc_embed_body(x_ref, zemb_ref, ew_ref, eb_ref, deg_ref, w0_ref,
                   h_ref, dinv_ref, hw_ref):
    x = x_ref[...]
    z = x[:, 0:1].astype(jnp.int32)
    onehot = (z == lax.broadcasted_iota(jnp.int32, (N, NA), 1)).astype(jnp.float32)
    h = jnp.dot(onehot, zemb_ref[...], preferred_element_type=jnp.float32)
    h = h + jnp.dot(x[:, 1:], ew_ref[...].T, preferred_element_type=jnp.float32)
    h = h + eb_ref[...]
    deg = deg_ref[0:N, 0:1] + deg_ref[DEG_PAD:DEG_PAD + N, 0:1] + 1.0
    dinv = lax.rsqrt(deg)
    h_ref[...] = h
    dinv_ref[...] = dinv
    hw = jnp.dot(h, w0_ref[...].T, preferred_element_type=jnp.float32) * dinv
    hw_ref[0, :N] = hw[:, :HHALF]
    hw_ref[1, :N] = hw[:, HHALF:]


def _layer_math(acc_ref, dinv_ref, hres_ref, batch_ref, cb_ref, gnw_ref,
                gnb_ref, gnms_ref, w1_ref, b1_ref, w2_ref, b2_ref):
    dinv = dinv_ref[...]
    hcat = jnp.concatenate([acc_ref[0, :N], acc_ref[1, :N]], axis=1)
    h1 = dinv * hcat + cb_ref[...]
    seg = (batch_ref[...].astype(jnp.int32)
           == lax.broadcasted_iota(jnp.int32, (G, N), 0))
    seg = seg.astype(jnp.float32)
    cnt = jnp.maximum(jnp.sum(seg, axis=1, keepdims=True), 1.0)
    mean = jnp.dot(seg, h1, preferred_element_type=jnp.float32) / cnt
    mean_n = lax.dot_general(seg, mean, (((0,), (0,)), ((), ())),
                             preferred_element_type=jnp.float32)
    out_c = h1 - mean_n * gnms_ref[...]
    var = jnp.dot(seg, out_c * out_c, preferred_element_type=jnp.float32) / cnt
    var_n = lax.dot_general(seg, var, (((0,), (0,)), ((), ())),
                            preferred_element_type=jnp.float32)
    h2 = gnw_ref[...] * out_c / jnp.sqrt(var_n + EPS) + gnb_ref[...]
    h3 = _swish(h2)
    m = _swish(jnp.dot(h3, w1_ref[...].T, preferred_element_type=jnp.float32)
               + b1_ref[...])
    m = _swish(jnp.dot(m, w2_ref[...].T, preferred_element_type=jnp.float32)
               + b2_ref[...])
    return m + hres_ref[...]


def _tc_layer_body(acc_ref, dinv_ref, hres_ref, batch_ref, cb_ref, gnw_ref,
                   gnb_ref, gnms_ref, w1_ref, b1_ref, w2_ref, b2_ref, wnext_ref,
                   h_ref, hw_ref):
    h_out = _layer_math(acc_ref, dinv_ref, hres_ref, batch_ref, cb_ref, gnw_ref,
                        gnb_ref, gnms_ref, w1_ref, b1_ref, w2_ref, b2_ref)
    h_ref[...] = h_out
    hw = jnp.dot(h_out, wnext_ref[...].T,
                 preferred_element_type=jnp.float32) * dinv_ref[...]
    hw_ref[0, :N] = hw[:, :HHALF]
    hw_ref[1, :N] = hw[:, HHALF:]


def _tc_final_body(acc_ref, dinv_ref, hres_ref, batch_ref, cb_ref, gnw_ref,
                   gnb_ref, gnms_ref, w1_ref, b1_ref, w2_ref, b2_ref, h_ref):
    h_ref[...] = _layer_math(acc_ref, dinv_ref, hres_ref, batch_ref, cb_ref,
                             gnw_ref, gnb_ref, gnms_ref, w1_ref, b1_ref,
                             w2_ref, b2_ref)


_f32 = jnp.float32

_tc_embed = pl.pallas_call(
    _tc_embed_body,
    out_shape=[
        jax.ShapeDtypeStruct((N, H), _f32),
        jax.ShapeDtypeStruct((N, 1), _f32),
        jax.ShapeDtypeStruct((2, NPAD, HHALF), _f32),
    ],
)

_tc_layer = pl.pallas_call(
    _tc_layer_body,
    out_shape=[
        jax.ShapeDtypeStruct((N, H), _f32),
        jax.ShapeDtypeStruct((2, NPAD, HHALF), _f32),
    ],
)

_tc_final = pl.pallas_call(
    _tc_final_body,
    out_shape=jax.ShapeDtypeStruct((N, H), _f32),
)


def kernel(x, edge_index, batch, z_embed, extra_W, extra_b, conv_W, conv_b,
           gn_w, gn_b, gn_ms, mlp_W1, mlp_b1, mlp_W2, mlp_b2):
    src = edge_index[0]
    dst = edge_index[1]
    pad = E_PAD - E
    src_t = jnp.concatenate([src, jnp.zeros((pad,), jnp.int32)]).reshape(
        NTILES, NBODY, BODY, K)
    dst_t = jnp.concatenate([dst, jnp.full((pad,), N, jnp.int32)]).reshape(
        NTILES, NBODY, BODY, K)
    idx_t = jnp.stack([src_t, dst_t], axis=2)

    zeros_deg = jnp.zeros((DEG_PAD, 16), _f32)
    ones_k = jnp.ones((K, 16), _f32)
    deg16 = _sc_deg(dst_t.reshape(NTILES, CHUNKS, K), zeros_deg, ones_k)

    batch_row = batch.astype(_f32).reshape(1, N)
    h, dinv, hw = _tc_embed(x, z_embed, extra_W, extra_b.reshape(1, H),
                            deg16, conv_W[0])

    for i in range(L):
        acc = _sc_agg(hw.reshape(2 * NPAD, HHALF), idx_t)
        args = (acc.reshape(2, NPAD, HHALF), dinv, h, batch_row,
                conv_b[i].reshape(1, H), gn_w[i].reshape(1, H),
                gn_b[i].reshape(1, H), gn_ms[i].reshape(1, H),
                mlp_W1[i], mlp_b1[i].reshape(1, M),
                mlp_W2[i], mlp_b2[i].reshape(1, H))
        if i + 1 < L:
            h, hw = _tc_layer(*args, conv_W[i + 1])
        else:
            h = _tc_final(*args)
    return h
